# baseline (speedup 1.0000x reference)
import jax
import jax.numpy as jnp
from jax import lax
from jax.experimental import pallas as pl
from jax.experimental.pallas import tpu as pltpu
from jax.experimental.pallas import tpu_sc as plsc

B = 1024
T = 77
D = 768
NC = 2
NS = 16
NW = NC * NS
ROWS = B * T
PER_W = ROWS // NW
C = 16
NCH = PER_W // C
NBUF = 8
L = NBUF // 2
NPOS = 4
DV = D // 16


def _add_pos(buf, pos_v, local_t):
    pvec = tuple(pos_v[local_t, pl.ds(j * 16, 16)] for j in range(DV))

    def row(r, pv):
        for j in range(DV):
            plsc.addupdate(buf.at[r, pl.ds(j * 16, 16)], pv[j])
        return pv

    lax.fori_loop(0, C, row, pvec)


def _body(tok_hbm, posw_hbm, tab_hbm, out_hbm, idx_v, pos_v, *rest):
    bufs = rest[:NBUF]
    gsem = rest[NBUF:2 * NBUF]
    ssem = rest[2 * NBUF:]
    wid = lax.axis_index("s") * NC + lax.axis_index("c")
    base = wid * PER_W
    t_lo = base // B

    pltpu.sync_copy(tok_hbm.at[wid], idx_v)
    pltpu.sync_copy(posw_hbm.at[wid], pos_v)

    def g_start(k, b):
        pltpu.async_copy(tab_hbm.at[idx_v.at[k]], bufs[b], gsem[b])

    def g_wait(k, b):
        pltpu.make_async_copy(tab_hbm.at[idx_v.at[k]], bufs[b], gsem[b]).wait()

    def s_start(k, b):
        pltpu.async_copy(bufs[b], out_hbm.at[pl.ds(base + k * C, C)], ssem[b])

    def s_wait(k, b):
        pltpu.make_async_copy(
            bufs[b], out_hbm.at[pl.ds(base + k * C, C)], ssem[b]).wait()

    def process(k, b):
        g_wait(k, b)
        _add_pos(bufs[b], pos_v, (base + k * C) // B - t_lo)
        s_start(k, b)

    for m in range(L):
        g_start(m, m)

    def lap(p, carry):
        for b in range(NBUF):
            k = NBUF * p + b

            @pl.when(k + L < NCH)
            def _launch():
                @pl.when(k >= NBUF - L)
                def _reuse():
                    s_wait(k + L - NBUF, (b + L) % NBUF)
                g_start(k + L, (b + L) % NBUF)

            @pl.when(k < NCH)
            def _proc():
                process(k, b)
        return carry

    lax.fori_loop(0, (NCH + NBUF - 1) // NBUF, lap, 0)

    for j in range(NBUF):
        kk = NCH - NBUF + j
        s_wait(kk, kk % NBUF)


@jax.jit
def _emb(tok, posw, tab):
    mesh = plsc.VectorSubcoreMesh(core_axis_name="c", subcore_axis_name="s",
                                  num_cores=NC, num_subcores=NS)
    f = pl.kernel(
        _body,
        out_type=jax.ShapeDtypeStruct((ROWS, D), jnp.float32),
        mesh=mesh,
        scratch_types=(
            [pltpu.VMEM((NCH, C), jnp.int32),
             pltpu.VMEM((NPOS, D), jnp.float32)]
            + [pltpu.VMEM((C, D), jnp.float32)] * NBUF
            + [pltpu.SemaphoreType.DMA] * (2 * NBUF)
        ),
    )
    return f(tok, posw, tab)


def kernel(tokens, token_embedding, position_embedding):
    tok = tokens.T.astype(jnp.int32).reshape(NW, NCH, C)
    t_lo = (jnp.arange(NW, dtype=jnp.int32) * PER_W) // B
    pos_pad = jnp.concatenate(
        [position_embedding,
         jnp.zeros((NPOS - 1, D), position_embedding.dtype)], axis=0)
    posw = pos_pad[t_lo[:, None] + jnp.arange(NPOS, dtype=jnp.int32)[None, :]]
    out = _emb(tok, posw, token_embedding)
    return out.reshape(T, B, D).transpose(1, 0, 2)


# speedup vs baseline: 1.0127x; 1.0127x over previous
import jax
import jax.numpy as jnp
from jax import lax
from jax.experimental import pallas as pl
from jax.experimental.pallas import tpu as pltpu
from jax.experimental.pallas import tpu_sc as plsc

B = 1024
T = 77
D = 768
NC = 2
NS = 16
NW = NC * NS
ROWS = B * T
PER_W = ROWS // NW
C = 32
NCH = PER_W // C
NBUF = 4
L = NBUF // 2
NPOS = 4
DV = D // 16


def _add_pos(buf, pos_v, local_t):
    pvec = tuple(pos_v[local_t, pl.ds(j * 16, 16)] for j in range(DV))

    def row(r, pv):
        for j in range(DV):
            plsc.addupdate(buf.at[r, pl.ds(j * 16, 16)], pv[j])
        return pv

    lax.fori_loop(0, C, row, pvec)


def _body(tok_hbm, posw_hbm, tab_hbm, out_hbm, idx_v, pos_v, *rest):
    bufs = rest[:NBUF]
    gsem = rest[NBUF:2 * NBUF]
    ssem = rest[2 * NBUF:]
    wid = lax.axis_index("s") * NC + lax.axis_index("c")
    base = wid * PER_W
    t_lo = base // B

    pltpu.sync_copy(tok_hbm.at[wid], idx_v)
    pltpu.sync_copy(posw_hbm.at[wid], pos_v)

    def g_start(k, b):
        pltpu.async_copy(tab_hbm.at[idx_v.at[k]], bufs[b], gsem[b])

    def g_wait(k, b):
        pltpu.make_async_copy(tab_hbm.at[idx_v.at[k]], bufs[b], gsem[b]).wait()

    def s_start(k, b):
        pltpu.async_copy(bufs[b], out_hbm.at[pl.ds(base + k * C, C)], ssem[b])

    def s_wait(k, b):
        pltpu.make_async_copy(
            bufs[b], out_hbm.at[pl.ds(base + k * C, C)], ssem[b]).wait()

    def process(k, b):
        g_wait(k, b)
        _add_pos(bufs[b], pos_v, (base + k * C) // B - t_lo)
        s_start(k, b)

    for m in range(L):
        g_start(m, m)

    def lap(p, carry):
        for b in range(NBUF):
            k = NBUF * p + b

            @pl.when(k + L < NCH)
            def _launch():
                @pl.when(k >= NBUF - L)
                def _reuse():
                    s_wait(k + L - NBUF, (b + L) % NBUF)
                g_start(k + L, (b + L) % NBUF)

            @pl.when(k < NCH)
            def _proc():
                process(k, b)
        return carry

    lax.fori_loop(0, (NCH + NBUF - 1) // NBUF, lap, 0)

    for j in range(NBUF):
        kk = NCH - NBUF + j
        s_wait(kk, kk % NBUF)


@jax.jit
def _emb(tok, posw, tab):
    mesh = plsc.VectorSubcoreMesh(core_axis_name="c", subcore_axis_name="s",
                                  num_cores=NC, num_subcores=NS)
    f = pl.kernel(
        _body,
        out_type=jax.ShapeDtypeStruct((ROWS, D), jnp.float32),
        mesh=mesh,
        scratch_types=(
            [pltpu.VMEM((NCH, C), jnp.int32),
             pltpu.VMEM((NPOS, D), jnp.float32)]
            + [pltpu.VMEM((C, D), jnp.float32)] * NBUF
            + [pltpu.SemaphoreType.DMA] * (2 * NBUF)
        ),
    )
    return f(tok, posw, tab)


def kernel(tokens, token_embedding, position_embedding):
    tok = tokens.T.astype(jnp.int32).reshape(NW, NCH, C)
    t_lo = (jnp.arange(NW, dtype=jnp.int32) * PER_W) // B
    pos_pad = jnp.concatenate(
        [position_embedding,
         jnp.zeros((NPOS - 1, D), position_embedding.dtype)], axis=0)
    posw = pos_pad[t_lo[:, None] + jnp.arange(NPOS, dtype=jnp.int32)[None, :]]
    out = _emb(tok, posw, token_embedding)
    return out.reshape(T, B, D).transpose(1, 0, 2)
